# Initial kernel scaffold; baseline (speedup 1.0000x reference)
import jax
import jax.numpy as jnp
from jax.experimental import pallas as pl


def kernel(x, adj, branch_temps, fusion_logits):
    raise NotImplementedError("write your pallas kernel here")


# speedup vs baseline: 5.7065x; 5.7065x over previous
import functools

import jax
import jax.numpy as jnp
from jax.experimental import pallas as pl
from jax.experimental.pallas import tpu as pltpu

N_NODES = 62
N_PAD = 64
N_LAYERS = 2
NUM_BRANCHES = 3
TOPK_START = 10
TOPK_END = 3
NEG = -1e30


def _adj_kernel(adj_ref, out_ref):
    a = adj_ref[...]
    rows = jax.lax.broadcasted_iota(jnp.int32, (N_PAD, N_PAD), 0)
    cols = jax.lax.broadcasted_iota(jnp.int32, (N_PAD, N_PAD), 1)
    valid = (rows < N_NODES) & (cols < N_NODES)
    eye = (rows == cols).astype(jnp.float32)
    a = jnp.clip(a, 0.0, 1.0) + eye
    a = jnp.maximum(a, 1e-8)
    a = jnp.where(valid, a, 0.0)
    row_sum = jnp.maximum(jnp.sum(a, axis=1, keepdims=True), 1e-8)
    d = jnp.clip(jax.lax.rsqrt(row_sum), 0.0, 100.0)
    rvalid = rows < N_NODES
    d = jnp.where(rvalid[:, :1], d, 0.0)
    out_ref[...] = (d * a) * d.reshape(1, N_PAD)


def _net_kernel(x_ref, adjn_ref, invt_ref, alpha_ref, out_ref, *, bs):
    adjn = adjn_ref[...]
    cols = jax.lax.broadcasted_iota(jnp.int32, (N_PAD, N_PAD), 1)
    rowsi = jax.lax.broadcasted_iota(jnp.int32, (N_PAD, N_PAD), 0)
    col_ok = cols < N_NODES
    diag = ((rowsi == cols) & col_ok).astype(jnp.float32) * 0.1

    for s in range(bs):
        x0 = x_ref[s]
        cur = x0
        for layer in range(N_LAYERS):
            k_top = int(TOPK_START - (TOPK_START - TOPK_END)
                        * (layer / max(1, N_LAYERS - 1)))
            nrm = jnp.sqrt(jnp.sum(cur * cur, axis=1, keepdims=True)) + 1e-6
            xn = cur / nrm
            sim = jax.lax.dot_general(
                xn, xn, (((1,), (1,)), ((), ())),
                preferred_element_type=jnp.float32)
            sim = sim * adjn
            y = None
            for b in range(NUM_BRANCHES):
                att = sim * invt_ref[b] + diag
                att_sel = jnp.where(col_ok, att, NEG)
                tmp = att_sel
                kth = None
                for _ in range(k_top):
                    kth = jnp.max(tmp, axis=1, keepdims=True)
                    tmp = jnp.where(tmp >= kth, NEG, tmp)
                att_m = jnp.where(att_sel >= kth, att, 0.0)
                att_m = jnp.where(col_ok, att_m, NEG)
                mx = jnp.max(att_m, axis=1, keepdims=True)
                e = jnp.exp(att_m - mx)
                p = e / jnp.sum(e, axis=1, keepdims=True)
                yk = jax.lax.dot_general(
                    p, cur, (((1,), (0,)), ((), ())),
                    preferred_element_type=jnp.float32)
                contrib = alpha_ref[layer, b] * yk
                y = contrib if y is None else y + contrib
            cur = y
            if layer > 0:
                cur = cur + x0
            if layer < N_LAYERS - 1:
                cur = jnp.maximum(cur, 0.0)
        out_ref[s] = cur


@jax.jit
def kernel(x, adj, branch_temps, fusion_logits):
    B, T, N, C = x.shape
    TC = T * C
    xf = jnp.transpose(x, (0, 2, 1, 3)).reshape(B, N, TC)
    xp = jnp.pad(xf, ((0, 0), (0, N_PAD - N), (0, 0)))

    adj_p = jnp.pad(adj, ((0, N_PAD - N), (0, N_PAD - N)))
    adjn_p = pl.pallas_call(
        _adj_kernel,
        out_shape=jax.ShapeDtypeStruct((N_PAD, N_PAD), jnp.float32),
    )(adj_p)

    inv_t = 1.0 / jnp.clip(branch_temps, 0.1, 10.0)
    alpha = jax.nn.softmax(fusion_logits, axis=-1)

    bs = 4
    out = pl.pallas_call(
        functools.partial(_net_kernel, bs=bs),
        grid=(B // bs,),
        in_specs=[
            pl.BlockSpec((bs, N_PAD, TC), lambda i: (i, 0, 0)),
            pl.BlockSpec((N_PAD, N_PAD), lambda i: (0, 0)),
            pl.BlockSpec(memory_space=pltpu.SMEM),
            pl.BlockSpec(memory_space=pltpu.SMEM),
        ],
        out_specs=pl.BlockSpec((bs, N_PAD, TC), lambda i: (i, 0, 0)),
        out_shape=jax.ShapeDtypeStruct((B, N_PAD, TC), jnp.float32),
    )(xp, adjn_p, inv_t, alpha)

    return (out[:, :N, :], adjn_p[:N, :N])


# speedup vs baseline: 6.9860x; 1.2242x over previous
import functools

import jax
import jax.numpy as jnp
from jax.experimental import pallas as pl
from jax.experimental.pallas import tpu as pltpu

N_NODES = 62
N_PAD = 64
N_LAYERS = 2
NUM_BRANCHES = 3
TOPK_START = 10
TOPK_END = 3
NEG = -1e30


def _adj_kernel(adj_ref, out_ref):
    a = adj_ref[...]
    rows = jax.lax.broadcasted_iota(jnp.int32, (N_PAD, N_PAD), 0)
    cols = jax.lax.broadcasted_iota(jnp.int32, (N_PAD, N_PAD), 1)
    valid = (rows < N_NODES) & (cols < N_NODES)
    eye = (rows == cols).astype(jnp.float32)
    a = jnp.clip(a, 0.0, 1.0) + eye
    a = jnp.maximum(a, 1e-8)
    a = jnp.where(valid, a, 0.0)
    row_sum = jnp.maximum(jnp.sum(a, axis=1, keepdims=True), 1e-8)
    d = jnp.clip(jax.lax.rsqrt(row_sum), 0.0, 100.0)
    rvalid = rows < N_NODES
    d = jnp.where(rvalid[:, :1], d, 0.0)
    out_ref[...] = (d * a) * d.reshape(1, N_PAD)


def _net_kernel(x_ref, adjn_ref, invt_ref, alpha_ref, out_ref, *, bs):
    adjn = adjn_ref[...]
    cols = jax.lax.broadcasted_iota(jnp.int32, (N_PAD, N_PAD), 1)
    rowsi = jax.lax.broadcasted_iota(jnp.int32, (N_PAD, N_PAD), 0)
    col_ok = cols < N_NODES
    diag = ((rowsi == cols) & col_ok).astype(jnp.float32) * 0.1

    for s in range(bs):
        x0 = x_ref[s]
        cur = x0
        for layer in range(N_LAYERS):
            k_top = int(TOPK_START - (TOPK_START - TOPK_END)
                        * (layer / max(1, N_LAYERS - 1)))
            nrm = jnp.sqrt(jnp.sum(cur * cur, axis=1, keepdims=True)) + 1e-6
            xn = cur / nrm
            sim = jax.lax.dot_general(
                xn, xn, (((1,), (1,)), ((), ())),
                preferred_element_type=jnp.float32)
            sim = sim * adjn
            p_acc = None
            for b in range(NUM_BRANCHES):
                att = sim * invt_ref[b] + diag
                att_sel = jnp.where(col_ok, att, NEG)
                tmp = att_sel
                kth = None
                for _ in range(k_top):
                    kth = jnp.max(tmp, axis=1, keepdims=True)
                    tmp = jnp.where(tmp >= kth, NEG, tmp)
                att_m = jnp.where(att_sel >= kth, att, 0.0)
                att_m = jnp.where(col_ok, att_m, NEG)
                mx = jnp.max(att_m, axis=1, keepdims=True)
                e = jnp.exp(att_m - mx)
                p = (alpha_ref[layer, b] / jnp.sum(e, axis=1, keepdims=True)) * e
                p_acc = p if p_acc is None else p_acc + p
            cur = jax.lax.dot_general(
                p_acc, cur, (((1,), (0,)), ((), ())),
                preferred_element_type=jnp.float32)
            if layer > 0:
                cur = cur + x0
            if layer < N_LAYERS - 1:
                cur = jnp.maximum(cur, 0.0)
        out_ref[s] = cur


@jax.jit
def kernel(x, adj, branch_temps, fusion_logits):
    B, T, N, C = x.shape
    TC = T * C
    xf = jnp.transpose(x, (0, 2, 1, 3)).reshape(B, N, TC)
    xp = jnp.pad(xf, ((0, 0), (0, N_PAD - N), (0, 0)))

    adj_p = jnp.pad(adj, ((0, N_PAD - N), (0, N_PAD - N)))
    adjn_p = pl.pallas_call(
        _adj_kernel,
        out_shape=jax.ShapeDtypeStruct((N_PAD, N_PAD), jnp.float32),
    )(adj_p)

    inv_t = 1.0 / jnp.clip(branch_temps, 0.1, 10.0)
    alpha = jax.nn.softmax(fusion_logits, axis=-1)

    bs = 4
    out = pl.pallas_call(
        functools.partial(_net_kernel, bs=bs),
        grid=(B // bs,),
        in_specs=[
            pl.BlockSpec((bs, N_PAD, TC), lambda i: (i, 0, 0)),
            pl.BlockSpec((N_PAD, N_PAD), lambda i: (0, 0)),
            pl.BlockSpec(memory_space=pltpu.SMEM),
            pl.BlockSpec(memory_space=pltpu.SMEM),
        ],
        out_specs=pl.BlockSpec((bs, N_PAD, TC), lambda i: (i, 0, 0)),
        out_shape=jax.ShapeDtypeStruct((B, N_PAD, TC), jnp.float32),
    )(xp, adjn_p, inv_t, alpha)

    return (out[:, :N, :], adjn_p[:N, :N])
